# Initial kernel scaffold; baseline (speedup 1.0000x reference)
import jax
import jax.numpy as jnp
from jax.experimental import pallas as pl


def kernel(theta, inputs_embedding, inputs_word_embedding, topic_table, word_table, W1, b1, gamma, beta, W2, b2):
    raise NotImplementedError("write your pallas kernel here")


# speedup vs baseline: 33.2713x; 33.2713x over previous
import functools

import jax
import jax.numpy as jnp
from jax import lax
from jax.experimental import pallas as pl
from jax.experimental.pallas import tpu as pltpu
from jax.experimental.pallas import tpu_sc as plsc

_B, _NT, _HID, _V, _EMB = 4096, 256, 1024, 1995, 300
_VP = 2048


def _make_bag_scatter(B, W, Kp, CH):
    try:
        info = plsc.get_sparse_core_info()
        NC, NS, L = info.num_cores, info.num_subcores, info.num_lanes
    except ValueError:
        NC, NS, L = 2, 16, 16
    NW = NC * NS
    RPW = B // NW
    NCHUNK = RPW // CH
    NFULL = W // L
    REM = W - NFULL * L
    SW = (NFULL + (1 if REM else 0)) * L
    UN = 4
    assert B % NW == 0 and RPW % CH == 0 and NFULL % UN == 0 and Kp % L == 0

    mesh = plsc.VectorSubcoreMesh(core_axis_name="c", subcore_axis_name="s",
                                  num_cores=NC, num_subcores=NS)

    @functools.partial(
        pl.kernel,
        out_type=jax.ShapeDtypeStruct((B, Kp), jnp.float32),
        mesh=mesh,
        compiler_params=pltpu.CompilerParams(needs_layout_passes=False,
                                             use_tc_tiling_on_sc=False),
        scratch_types=[
            pltpu.VMEM((CH * SW,), jnp.float32),
            pltpu.VMEM((CH * SW,), jnp.int32),
            pltpu.VMEM((CH * Kp,), jnp.float32),
        ],
    )
    def bag(w_hbm, i_hbm, out_hbm, wbuf, ibuf, acc):
        wid = lax.axis_index("s") * NC + lax.axis_index("c")

        def chunk(c, carry):
            rowbase = wid * RPW + c * CH
            for r in range(CH):
                pltpu.sync_copy(w_hbm.at[rowbase + r], wbuf.at[pl.ds(r * SW, W)])
                pltpu.sync_copy(i_hbm.at[rowbase + r], ibuf.at[pl.ds(r * SW, W)])

            def zbody(k, carry2):
                for u in range(UN):
                    acc[pl.ds((k * UN + u) * L, L)] = jnp.zeros((L,), jnp.float32)
                return carry2

            lax.fori_loop(0, CH * Kp // (L * UN), zbody, 0)

            for r in range(CH):
                roff = r * SW
                tgt = r * Kp

                def gbody(j, carry2, roff=roff, tgt=tgt):
                    for u in range(UN):
                        off = roff + (j * UN + u) * L
                        iv = ibuf[pl.ds(off, L)] + tgt
                        wv = wbuf[pl.ds(off, L)]
                        plsc.addupdate_scatter(acc, [iv], wv)
                    return carry2

                lax.fori_loop(0, NFULL // UN, gbody, 0)
                if REM:
                    valid = lax.iota(jnp.int32, L) < REM
                    toff = roff + NFULL * L
                    iv = jnp.where(valid, ibuf[pl.ds(toff, L)], 0) + tgt
                    wv = jnp.where(valid, wbuf[pl.ds(toff, L)], 0.0)
                    plsc.addupdate_scatter(acc, [iv], wv)

            for r in range(CH):
                pltpu.sync_copy(acc.at[pl.ds(r * Kp, Kp)],
                                out_hbm.at[rowbase + r])
            return carry

        lax.fori_loop(0, NCHUNK, chunk, 0)

    return bag


def _stats_label_kernel(theta_ref, w1_ref, b1_ref, stats_ref, lab_ref):
    i = pl.program_id(0)
    th = theta_ref[...]
    h = jnp.dot(th, w1_ref[...], preferred_element_type=jnp.float32) + b1_ref[...]
    part = jnp.concatenate(
        [jnp.sum(h, axis=0, keepdims=True), jnp.sum(h * h, axis=0, keepdims=True)],
        axis=0,
    )

    @pl.when(i == 0)
    def _():
        stats_ref[...] = part

    @pl.when(i != 0)
    def _():
        stats_ref[...] = stats_ref[...] + part

    m = jnp.max(th, axis=1, keepdims=True)
    col = lax.broadcasted_iota(jnp.int32, th.shape, 1)
    lab = jnp.min(jnp.where(th == m, col, _NT), axis=1)
    lab_ref[...] = lab[:, None]


def _bow_kernel(theta_ref, w1_ref, b1_ref, stats_ref, g_ref, be_ref, w2_ref,
                b2_ref, bow_ref):
    th = theta_ref[...]
    h = jnp.dot(th, w1_ref[...], preferred_element_type=jnp.float32) + b1_ref[...]
    mean = stats_ref[0:1, :] * (1.0 / _B)
    var = stats_ref[1:2, :] * (1.0 / _B) - mean * mean
    hn = (h - mean) / jnp.sqrt(var + 1e-5) * g_ref[...] + be_ref[...]
    hl = jnp.where(hn >= 0, hn, 0.01 * hn)
    logits = jnp.dot(hl, w2_ref[...], preferred_element_type=jnp.float32) + b2_ref[...]
    mx = jnp.max(logits, axis=1, keepdims=True)
    e = jnp.exp(logits - mx)
    p = e / jnp.sum(e, axis=1, keepdims=True)
    bow_ref[...] = p[:, :_V]


def _emb_kernel(s2_ref, wt_ref, s1_ref, tt_ref, z_ref, te_ref):
    z_ref[...] = jnp.dot(s2_ref[...], wt_ref[...], preferred_element_type=jnp.float32)
    te_ref[...] = jnp.dot(s1_ref[...], tt_ref[...], preferred_element_type=jnp.float32)


def kernel(theta, inputs_embedding, inputs_word_embedding, topic_table,
           word_table, W1, b1, gamma, beta, W2, b2):
    f32 = jnp.float32
    pad = _VP - _V
    W2p = jnp.pad(W2, ((0, 0), (0, pad)))
    b2p = jnp.pad(b2, (0, pad), constant_values=-1e30)
    wtp = jnp.pad(word_table, ((0, pad), (0, 0)))

    BBA, BBB, BBC = 512, 256, 512

    stats, lab = pl.pallas_call(
        _stats_label_kernel,
        grid=(_B // BBA,),
        in_specs=[
            pl.BlockSpec((BBA, _NT), lambda i: (i, 0)),
            pl.BlockSpec((_NT, _HID), lambda i: (0, 0)),
            pl.BlockSpec((1, _HID), lambda i: (0, 0)),
        ],
        out_specs=[
            pl.BlockSpec((2, _HID), lambda i: (0, 0)),
            pl.BlockSpec((BBA, 1), lambda i: (i, 0)),
        ],
        out_shape=[
            jax.ShapeDtypeStruct((2, _HID), f32),
            jax.ShapeDtypeStruct((_B, 1), jnp.int32),
        ],
    )(theta, W1, b1[None, :])

    bow = pl.pallas_call(
        _bow_kernel,
        grid=(_B // BBB,),
        in_specs=[
            pl.BlockSpec((BBB, _NT), lambda i: (i, 0)),
            pl.BlockSpec((_NT, _HID), lambda i: (0, 0)),
            pl.BlockSpec((1, _HID), lambda i: (0, 0)),
            pl.BlockSpec((2, _HID), lambda i: (0, 0)),
            pl.BlockSpec((1, _HID), lambda i: (0, 0)),
            pl.BlockSpec((1, _HID), lambda i: (0, 0)),
            pl.BlockSpec((_HID, _VP), lambda i: (0, 0)),
            pl.BlockSpec((1, _VP), lambda i: (0, 0)),
        ],
        out_specs=pl.BlockSpec((BBB, _V), lambda i: (i, 0)),
        out_shape=jax.ShapeDtypeStruct((_B, _V), f32),
    )(theta, W1, b1[None, :], stats, gamma[None, :], beta[None, :], W2p,
      b2p[None, :])

    s1 = _make_bag_scatter(_B, _NT, _NT, 16)(theta, inputs_embedding)
    s2 = _make_bag_scatter(_B, _V, _VP, 8)(bow, inputs_word_embedding)

    z, te = pl.pallas_call(
        _emb_kernel,
        grid=(_B // BBC,),
        in_specs=[
            pl.BlockSpec((BBC, _VP), lambda i: (i, 0)),
            pl.BlockSpec((_VP, _EMB), lambda i: (0, 0)),
            pl.BlockSpec((BBC, _NT), lambda i: (i, 0)),
            pl.BlockSpec((_NT, _EMB), lambda i: (0, 0)),
        ],
        out_specs=[
            pl.BlockSpec((BBC, _EMB), lambda i: (i, 0)),
            pl.BlockSpec((BBC, _EMB), lambda i: (i, 0)),
        ],
        out_shape=[
            jax.ShapeDtypeStruct((_B, _EMB), f32),
            jax.ShapeDtypeStruct((_B, _EMB), f32),
        ],
    )(s2, wtp, s1, topic_table)

    return (bow, te, z, lab.reshape(_B))
